# baseline (speedup 1.0000x reference)
import functools

import jax
import jax.numpy as jnp
from jax import lax
from jax.experimental import pallas as pl
from jax.experimental.pallas import tpu as pltpu
from jax.experimental.pallas import tpu_sc as plsc

_SPLITS = (1021, 1023, 1024, 1019, 1025, 1022, 1026, 1024)
_ALIGN = 16
_F = 2048
_PADDED = tuple((m + _ALIGN - 1) // _ALIGN * _ALIGN for m in _SPLITS)
_TOTAL_IN = sum(_SPLITS)
_TOTAL_OUT = sum(_PADDED)
_NW = 32
_WPE = _NW // len(_SPLITS)


def _worker_jobs():
    src_off, dst_off = [], []
    s = d = 0
    for m, pm in zip(_SPLITS, _PADDED):
        src_off.append(s)
        dst_off.append(d)
        s += m
        d += pm
    jobs = []
    for w in range(_NW):
        e, q = divmod(w, _WPE)
        m, pm = _SPLITS[e], _PADDED[e]
        rpq = pm // _WPE
        lo, hi = q * rpq, (q + 1) * rpq
        n_copy = max(0, min(hi, m) - lo)
        copies = []
        if n_copy:
            copies.append((src_off[e] + lo, dst_off[e] + lo, n_copy))
        zero_rows = list(range(dst_off[e] + max(lo, m), dst_off[e] + hi))
        jobs.append((copies, zero_rows))
    return jobs


_JOBS = _worker_jobs()
_NEEDS_ZERO = [w for w, (_, z) in enumerate(_JOBS) if z]


@functools.partial(
    pl.kernel,
    mesh=plsc.VectorSubcoreMesh(core_axis_name="c", subcore_axis_name="s"),
    out_type=jax.ShapeDtypeStruct((_TOTAL_OUT * _F,), jnp.float32),
    scratch_types=[
        pltpu.VMEM((_F,), jnp.float32),
        pltpu.SemaphoreType.DMA,
    ],
)
def _pad_rows(inp_hbm, out_hbm, zrow, sem):
    wid = lax.axis_index("s") * 2 + lax.axis_index("c")

    zv = jnp.zeros((16,), jnp.float32)
    for i in range(_F // 16):
        zrow[pl.ds(i * 16, 16)] = zv

    for w, (copies, zero_rows) in enumerate(_JOBS):
        def _body(copies=copies, zero_rows=zero_rows):
            for (src, dst, n) in copies:
                pltpu.async_copy(
                    inp_hbm.at[pl.ds(src * _F, n * _F)],
                    out_hbm.at[pl.ds(dst * _F, n * _F)],
                    sem,
                ).wait()
            for r in zero_rows:
                pltpu.sync_copy(zrow, out_hbm.at[pl.ds(r * _F, _F)])

        pl.when(wid == w)(_body)


def kernel(inp, m_splits):
    out = _pad_rows(inp.reshape(-1)).reshape(_TOTAL_OUT, _F)
    deltas = jnp.array([pm - m for m, pm in zip(_SPLITS, _PADDED)],
                       dtype=jnp.int64)
    return out, jnp.asarray(m_splits, dtype=jnp.int64) + deltas


# speedup vs baseline: 10.7023x; 10.7023x over previous
import functools

import jax
import jax.numpy as jnp
from jax import lax
from jax.experimental import pallas as pl
from jax.experimental.pallas import tpu as pltpu
from jax.experimental.pallas import tpu_sc as plsc

_SPLITS = (1021, 1023, 1024, 1019, 1025, 1022, 1026, 1024)
_ALIGN = 16
_F = 2048
_PADDED = tuple((m + _ALIGN - 1) // _ALIGN * _ALIGN for m in _SPLITS)
_TOTAL_IN = sum(_SPLITS)
_TOTAL_OUT = sum(_PADDED)
_NW = 32
_WPE = _NW // len(_SPLITS)


def _worker_jobs():
    src_off, dst_off = [], []
    s = d = 0
    for m, pm in zip(_SPLITS, _PADDED):
        src_off.append(s)
        dst_off.append(d)
        s += m
        d += pm
    jobs = []
    for w in range(_NW):
        e, q = divmod(w, _WPE)
        m, pm = _SPLITS[e], _PADDED[e]
        rpq = pm // _WPE
        lo, hi = q * rpq, (q + 1) * rpq
        n_copy = max(0, min(hi, m) - lo)
        copies = []
        if n_copy:
            copies.append((src_off[e] + lo, dst_off[e] + lo, n_copy))
        zero_rows = list(range(dst_off[e] + max(lo, m), dst_off[e] + hi))
        jobs.append((copies, zero_rows))
    return jobs


_JOBS = _worker_jobs()
_NEEDS_ZERO = [w for w, (_, z) in enumerate(_JOBS) if z]


_CH = 30


def _chunks(src, dst, n):
    pieces = []
    off = 0
    while off < n:
        c = min(_CH, n - off)
        pieces.append((src + off, dst + off, c))
        off += c
    return pieces


@functools.partial(
    pl.kernel,
    mesh=plsc.VectorSubcoreMesh(core_axis_name="c", subcore_axis_name="s"),
    out_type=jax.ShapeDtypeStruct((_TOTAL_OUT * _F,), jnp.float32),
    scratch_types=[
        pltpu.VMEM((2, _CH * _F), jnp.float32),
        pltpu.VMEM((_F,), jnp.float32),
        pltpu.SemaphoreType.DMA,
        pltpu.SemaphoreType.DMA,
        pltpu.SemaphoreType.DMA,
        pltpu.SemaphoreType.DMA,
    ],
)
def _pad_rows(inp_hbm, out_hbm, buf, zrow, g0, g1, s0, s1):
    wid = lax.axis_index("s") * 2 + lax.axis_index("c")
    gsem = (g0, g1)
    ssem = (s0, s1)

    zv = jnp.zeros((16,), jnp.float32)
    for i in range(_F // 16):
        zrow[pl.ds(i * 16, 16)] = zv

    def _gather(piece, b):
        src, _, c = piece
        return pltpu.async_copy(
            inp_hbm.at[pl.ds(src * _F, c * _F)],
            buf.at[b, pl.ds(0, c * _F)],
            gsem[b],
        )

    def _scatter(piece, b):
        _, dst, c = piece
        return pltpu.async_copy(
            buf.at[b, pl.ds(0, c * _F)],
            out_hbm.at[pl.ds(dst * _F, c * _F)],
            ssem[b],
        )

    for w, (copies, zero_rows) in enumerate(_JOBS):
        def _body(copies=copies, zero_rows=zero_rows):
            for r in zero_rows:
                pltpu.sync_copy(zrow, out_hbm.at[pl.ds(r * _F, _F)])
            for (src, dst, n) in copies:
                pieces = _chunks(src, dst, n)
                np_ = len(pieces)
                gathers = [None] * np_
                scatters = [None] * np_
                gathers[0] = _gather(pieces[0], 0)
                for i in range(np_):
                    b = i % 2
                    if i + 1 < np_:
                        if i >= 1:
                            scatters[i - 1].wait()
                        gathers[i + 1] = _gather(pieces[i + 1], 1 - b)
                    gathers[i].wait()
                    scatters[i] = _scatter(pieces[i], b)
                if np_ >= 2:
                    scatters[np_ - 2].wait()
                scatters[np_ - 1].wait()

        pl.when(wid == w)(_body)


def kernel(inp, m_splits):
    out = _pad_rows(inp.reshape(-1)).reshape(_TOTAL_OUT, _F)
    deltas = jnp.array([pm - m for m, pm in zip(_SPLITS, _PADDED)],
                       dtype=jnp.int64)
    return out, jnp.asarray(m_splits, dtype=jnp.int64) + deltas


# speedup vs baseline: 31.4446x; 2.9381x over previous
import functools

import jax
import jax.numpy as jnp
from jax import lax
from jax.experimental import pallas as pl
from jax.experimental.pallas import tpu as pltpu
from jax.experimental.pallas import tpu_sc as plsc

_SPLITS = (1021, 1023, 1024, 1019, 1025, 1022, 1026, 1024)
_ALIGN = 16
_F = 2048
_PADDED = tuple((m + _ALIGN - 1) // _ALIGN * _ALIGN for m in _SPLITS)
_TOTAL_IN = sum(_SPLITS)
_TOTAL_OUT = sum(_PADDED)
_NW = 32
_WPE = _NW // len(_SPLITS)

_CH = 16


def _worker_params():
    src_off, dst_off = [], []
    s = d = 0
    for m, pm in zip(_SPLITS, _PADDED):
        src_off.append(s)
        dst_off.append(d)
        s += m
        d += pm
    n_l, src0_l, dst0_l, pad0_l, pend_l = [], [], [], [], []
    for w in range(_NW):
        e, q = divmod(w, _WPE)
        m, pm = _SPLITS[e], _PADDED[e]
        base = (pm // _WPE) // 16 * 16
        sizes = [base] * _WPE
        sizes[0] += pm - base * _WPE
        lo = dst_off[e] + sum(sizes[:q])
        shift = src_off[e] - dst_off[e]
        n_l.append(sizes[q] // _CH)
        src0_l.append(lo + shift)
        dst0_l.append(lo)
        has_pad = q == _WPE - 1 and pm != m
        pad0_l.append(dst_off[e] + m if has_pad else -1)
        pend_l.append(dst_off[e] + pm)
    return n_l, src0_l, dst0_l, pad0_l, pend_l


_N_L, _SRC0_L, _DST0_L, _PAD0_L, _PEND_L = _worker_params()


@functools.partial(
    pl.kernel,
    mesh=plsc.VectorSubcoreMesh(core_axis_name="c", subcore_axis_name="s"),
    out_type=jax.ShapeDtypeStruct((_TOTAL_OUT, _F), jnp.float32),
    scratch_types=[
        pltpu.VMEM((2, _CH, _F), jnp.float32),
        pltpu.SemaphoreType.DMA,
        pltpu.SemaphoreType.DMA,
        pltpu.SemaphoreType.DMA,
        pltpu.SemaphoreType.DMA,
    ],
)
def _pad_rows(inp_hbm, zeros_hbm, out_hbm, buf, g0, g1, s0, s1):
    wid = lax.axis_index("s") * 2 + lax.axis_index("c")
    gsem = (g0, g1)
    ssem = (s0, s1)
    iota = lax.iota(jnp.int32, 16)

    def _sel(vals):
        x = jnp.int32(vals[0])
        for w in range(1, _NW):
            x = jnp.where(wid == w, jnp.int32(vals[w]), x)
        return x

    n = _sel(_N_L)
    src0 = _sel(_SRC0_L)
    dst0 = _sel(_DST0_L)
    pad0 = _sel(_PAD0_L)
    pend = _sel(_PEND_L)

    def _g_start(i, b):
        pltpu.async_copy(inp_hbm.at[iota + (src0 + _CH * i)], buf.at[b],
                         gsem[b])

    def _g_wait(b):
        pltpu.make_async_copy(inp_hbm.at[pl.ds(0, _CH)], buf.at[b],
                              gsem[b]).wait()

    def _s_start(i, b):
        dst = pl.multiple_of(dst0 + _CH * i, 8)
        pltpu.async_copy(buf.at[b], out_hbm.at[pl.ds(dst, _CH)], ssem[b])

    def _s_wait(b):
        pltpu.make_async_copy(buf.at[b], out_hbm.at[pl.ds(0, _CH)],
                              ssem[b]).wait()

    _g_start(jnp.int32(0), 0)
    nhalf = (n + 1) // 2

    def _pair(j, carry):
        i0 = 2 * j
        i1 = i0 + 1

        @pl.when(i1 < n)
        def _():
            @pl.when(j >= 1)
            def _():
                _s_wait(1)
            _g_start(i1, 1)

        _g_wait(0)
        _s_start(i0, 0)

        @pl.when(i1 < n)
        def _():
            _g_wait(1)

            @pl.when(i1 + 1 < n)
            def _():
                _s_wait(0)
                _g_start(i1 + 1, 0)

            _s_start(i1, 1)

        return carry

    lax.fori_loop(0, nhalf, _pair, 0)
    _s_wait(0)
    _s_wait(1)

    @pl.when(pad0 >= 0)
    def _():
        pltpu.async_copy(zeros_hbm, buf.at[0], gsem[0]).wait()
        zidx = jnp.minimum(iota + pad0, pend - 1)
        pltpu.async_copy(buf.at[0], out_hbm.at[zidx], ssem[0]).wait()


def kernel(inp, m_splits):
    zeros = jnp.zeros((_CH, _F), dtype=inp.dtype)
    out = _pad_rows(inp, zeros)
    deltas = jnp.array([pm - m for m, pm in zip(_SPLITS, _PADDED)],
                       dtype=jnp.int64)
    return out, jnp.asarray(m_splits, dtype=jnp.int64) + deltas
